# baseline (speedup 1.0000x reference)
import jax
import jax.numpy as jnp
from jax import lax
from jax.experimental import pallas as pl
from jax.experimental.pallas import tpu as pltpu

B, Q, T, H, W = 4, 128, 64, 64, 64
HW = H * W
WIN = 7
NWIN = WIN * WIN
HALF = WIN // 2
C_OCC = 8
NO_ELECTRON_WEIGHT = 0.1
LOG_2PI = 1.8378770664093453
NROWS = 9


def _softplus(x):
    return jnp.maximum(x, 0.0) + jnp.log1p(jnp.exp(-jnp.abs(x)))


def _loss_kernel(portion_ref, binary_ref, true_ref, small_ref,
                 occ_ref, occ_tgt_ref, out_ref, acc_ref):
    b = pl.program_id(0)

    matched = small_ref[0, 1:2, 0:T].astype(jnp.int32)
    q_iota = lax.broadcasted_iota(jnp.int32, (Q, T), 0)
    onehot = (q_iota == matched).astype(jnp.float32)

    true_b = true_ref[0].reshape(HW, T)

    rp = lax.dot_general(
        portion_ref[0].reshape(HW, Q), onehot, (((1,), (0,)), ((), ())),
        precision=lax.Precision.DEFAULT,
        preferred_element_type=jnp.float32)
    p = 0.5 * jnp.tanh(0.5 * rp) + 0.5
    num_t = 2.0 * jnp.sum(p * true_b, axis=0, keepdims=True)
    den_t = jnp.sum(p + true_b, axis=0, keepdims=True)
    dice_b = jnp.sum(1.0 - (num_t + 1.0) / (den_t + 1.0))

    rb = lax.dot_general(
        binary_ref[0].reshape(HW, Q), onehot, (((1,), (0,)), ((), ())),
        precision=lax.Precision.DEFAULT,
        preferred_element_type=jnp.float32)
    ix = small_ref[0, 2:3, 0:T]
    iy = small_ref[0, 3:4, 0:T]
    r_t = jnp.floor(ix).astype(jnp.int32)
    c_t = jnp.floor(iy).astype(jnp.int32)
    pix = lax.broadcasted_iota(jnp.int32, (HW, T), 0)
    a = pix + ((HALF * W + HALF) - (r_t * W + c_t))
    inwin = ((a & (W - 1)) <= 2 * HALF) & \
            (lax.shift_right_logical(a, 6) <= 2 * HALF)
    bce_el = _softplus(rb) - rb * true_b
    bce_b = jnp.sum(jnp.where(inwin, bce_el, 0.0))

    x_ie = small_ref[0, 0:1, :]
    ones_t = jnp.ones((1, T), jnp.float32)
    labels = lax.dot_general(
        ones_t, onehot, (((1,), (1,)), ((), ())),
        precision=lax.Precision.DEFAULT,
        preferred_element_type=jnp.float32)
    wts = jnp.where(labels > 0.0, 1.0, NO_ELECTRON_WEIGHT)
    class_b = jnp.sum(wts * (_softplus(x_ie) - x_ie * labels))

    g = lax.dot_general(
        small_ref[0, 4:9, :], onehot, (((1,), (0,)), ((), ())),
        precision=lax.Precision.HIGHEST,
        preferred_element_type=jnp.float32)
    d0 = ix - g[0:1, :]
    d1 = iy - g[1:2, :]
    l00 = g[2:3, :]
    l10 = g[3:4, :]
    l11 = g[4:5, :]
    z0 = d0 / l00
    z1 = (d1 - l10 * z0) / l11
    nll_b = jnp.sum(0.5 * (z0 * z0 + z1 * z1)
                    + jnp.log(jnp.abs(l00)) + jnp.log(jnp.abs(l11)) + LOG_2PI)

    @pl.when(b == 0)
    def _init():
        for i in range(4):
            acc_ref[i] = 0.0

    acc_ref[0] = acc_ref[0] + class_b
    acc_ref[1] = acc_ref[1] + bce_b
    acc_ref[2] = acc_ref[2] + dice_b
    acc_ref[3] = acc_ref[3] + nll_b

    @pl.when(b == B - 1)
    def _final():
        xo = occ_ref[:, :]
        m = jnp.max(xo, axis=1, keepdims=True)
        lse = m + jnp.log(jnp.sum(jnp.exp(xo - m), axis=1, keepdims=True))
        logp = xo - lse
        c_iota = lax.broadcasted_iota(jnp.int32, (1, C_OCC), 1)
        occ_sum = 0.0
        for i in range(B):
            sel = (c_iota == occ_tgt_ref[i]).astype(jnp.float32)
            occ_sum = occ_sum + jnp.sum(sel * logp[i:i + 1, :])
        out_ref[0] = (acc_ref[0] / (B * Q)
                      + acc_ref[1] / (B * T * NWIN)
                      + acc_ref[2] / (B * T)
                      + acc_ref[3] / (B * T)
                      - occ_sum / B)


@jax.jit
def kernel(is_electron_logit, positions, position_std_dev_cholesky, true_segmap,
           binary_mask_logits, portion_logits, occupancy_logits, incidence_points,
           matched_pred, occupancy_target):
    pad = jnp.zeros((B, Q - T), jnp.float32)
    small = jnp.stack([
        is_electron_logit.reshape(B, Q),
        jnp.concatenate([matched_pred.astype(jnp.float32), pad], axis=1),
        jnp.concatenate([incidence_points[..., 0], pad], axis=1),
        jnp.concatenate([incidence_points[..., 1], pad], axis=1),
        positions[:, 0].reshape(B, Q),
        positions[:, 1].reshape(B, Q),
        position_std_dev_cholesky[:, 0, 0].reshape(B, Q),
        position_std_dev_cholesky[:, 1, 0].reshape(B, Q),
        position_std_dev_cholesky[:, 1, 1].reshape(B, Q),
    ], axis=1)

    out = pl.pallas_call(
        _loss_kernel,
        grid=(B,),
        in_specs=[
            pl.BlockSpec((1, H, W, Q), lambda b: (b, 0, 0, 0)),
            pl.BlockSpec((1, H, W, Q), lambda b: (b, 0, 0, 0)),
            pl.BlockSpec((1, H, W, T), lambda b: (b, 0, 0, 0)),
            pl.BlockSpec((1, NROWS, Q), lambda b: (b, 0, 0)),
            pl.BlockSpec((B, C_OCC), lambda b: (0, 0)),
            pl.BlockSpec(memory_space=pltpu.SMEM),
        ],
        out_specs=pl.BlockSpec(memory_space=pltpu.SMEM),
        out_shape=jax.ShapeDtypeStruct((1,), jnp.float32),
        scratch_shapes=[pltpu.SMEM((8,), jnp.float32)],
    )(portion_logits, binary_mask_logits, true_segmap, small,
      occupancy_logits, occupancy_target)
    return out[0]


# speedup vs baseline: 1.2879x; 1.2879x over previous
import jax
import jax.numpy as jnp
from jax import lax
from jax.experimental import pallas as pl
from jax.experimental.pallas import tpu as pltpu

B, Q, T, H, W = 4, 128, 64, 64, 64
HW = H * W
WIN = 7
NWIN = WIN * WIN
HALF = WIN // 2
C_OCC = 8
NO_ELECTRON_WEIGHT = 0.1
LOG_2PI = 1.8378770664093453


def _softplus(x):
    return jnp.maximum(x, 0.0) + jnp.log1p(jnp.exp(-jnp.abs(x)))


def _loss_kernel(portion_ref, binary_ref, true_ref, matched_ref, inc_ref,
                 ie_ref, pos_ref, chol_ref, occ_ref, occ_tgt_ref, out_ref, acc_ref):
    b = pl.program_id(0)

    matched = matched_ref[0]
    q_iota = lax.broadcasted_iota(jnp.int32, (Q, T), 0)
    onehot = (q_iota == matched).astype(jnp.float32)

    true_b = true_ref[0].reshape(HW, T)

    rp = lax.dot_general(
        portion_ref[0].reshape(HW, Q), onehot, (((1,), (0,)), ((), ())),
        precision=lax.Precision.DEFAULT,
        preferred_element_type=jnp.float32)
    p = 0.5 * jnp.tanh(0.5 * rp) + 0.5
    num_t = 2.0 * jnp.sum(p * true_b, axis=0, keepdims=True)
    den_t = jnp.sum(p + true_b, axis=0, keepdims=True)
    dice_b = jnp.sum(1.0 - (num_t + 1.0) / (den_t + 1.0))

    rb = lax.dot_general(
        binary_ref[0].reshape(HW, Q), onehot, (((1,), (0,)), ((), ())),
        precision=lax.Precision.DEFAULT,
        preferred_element_type=jnp.float32)
    ix = inc_ref[0, 0:1, :]
    iy = inc_ref[0, 1:2, :]
    r_t = jnp.floor(ix).astype(jnp.int32)
    c_t = jnp.floor(iy).astype(jnp.int32)
    pix = lax.broadcasted_iota(jnp.int32, (HW, T), 0)
    a = pix + ((HALF * W + HALF) - (r_t * W + c_t))
    inwin = ((a & (W - 1)) <= 2 * HALF) & \
            (lax.shift_right_logical(a, 6) <= 2 * HALF)
    bce_el = _softplus(rb) - rb * true_b
    bce_b = jnp.sum(jnp.where(inwin, bce_el, 0.0))

    x_ie = ie_ref[0]
    ones_t = jnp.ones((1, T), jnp.float32)
    labels = lax.dot_general(
        ones_t, onehot, (((1,), (1,)), ((), ())),
        precision=lax.Precision.DEFAULT,
        preferred_element_type=jnp.float32)
    wts = jnp.where(labels > 0.0, 1.0, NO_ELECTRON_WEIGHT)
    class_b = jnp.sum(wts * (_softplus(x_ie) - x_ie * labels))

    packed = jnp.concatenate(
        [pos_ref[0:1, :], pos_ref[1:2, :],
         chol_ref[0, 0:1, :], chol_ref[1, 0:1, :], chol_ref[1, 1:2, :]],
        axis=0)
    g = lax.dot_general(
        packed, onehot, (((1,), (0,)), ((), ())),
        precision=lax.Precision.HIGHEST,
        preferred_element_type=jnp.float32)
    d0 = ix - g[0:1, :]
    d1 = iy - g[1:2, :]
    l00 = g[2:3, :]
    l10 = g[3:4, :]
    l11 = g[4:5, :]
    z0 = d0 / l00
    z1 = (d1 - l10 * z0) / l11
    nll_b = jnp.sum(0.5 * (z0 * z0 + z1 * z1)
                    + jnp.log(jnp.abs(l00)) + jnp.log(jnp.abs(l11)) + LOG_2PI)

    @pl.when(b == 0)
    def _init():
        for i in range(4):
            acc_ref[i] = 0.0

    acc_ref[0] = acc_ref[0] + class_b
    acc_ref[1] = acc_ref[1] + bce_b
    acc_ref[2] = acc_ref[2] + dice_b
    acc_ref[3] = acc_ref[3] + nll_b

    @pl.when(b == B - 1)
    def _final():
        xo = occ_ref[:, :]
        m = jnp.max(xo, axis=1, keepdims=True)
        lse = m + jnp.log(jnp.sum(jnp.exp(xo - m), axis=1, keepdims=True))
        logp = xo - lse
        c_iota = lax.broadcasted_iota(jnp.int32, (1, C_OCC), 1)
        occ_sum = 0.0
        for i in range(B):
            sel = (c_iota == occ_tgt_ref[i]).astype(jnp.float32)
            occ_sum = occ_sum + jnp.sum(sel * logp[i:i + 1, :])
        out_ref[0] = (acc_ref[0] / (B * Q)
                      + acc_ref[1] / (B * T * NWIN)
                      + acc_ref[2] / (B * T)
                      + acc_ref[3] / (B * T)
                      - occ_sum / B)


@jax.jit
def kernel(is_electron_logit, positions, position_std_dev_cholesky, true_segmap,
           binary_mask_logits, portion_logits, occupancy_logits, incidence_points,
           matched_pred, occupancy_target):
    matched3 = matched_pred.reshape(B, 1, T)
    inc_t = incidence_points.transpose(0, 2, 1)
    ie = is_electron_logit.reshape(B, 1, Q)
    pos_t = positions.transpose(1, 0)
    chol_t = position_std_dev_cholesky.transpose(1, 2, 0)

    out = pl.pallas_call(
        _loss_kernel,
        grid=(B,),
        in_specs=[
            pl.BlockSpec((1, H, W, Q), lambda b: (b, 0, 0, 0)),
            pl.BlockSpec((1, H, W, Q), lambda b: (b, 0, 0, 0)),
            pl.BlockSpec((1, H, W, T), lambda b: (b, 0, 0, 0)),
            pl.BlockSpec((1, 1, T), lambda b: (b, 0, 0)),
            pl.BlockSpec((1, 2, T), lambda b: (b, 0, 0)),
            pl.BlockSpec((1, 1, Q), lambda b: (b, 0, 0)),
            pl.BlockSpec((2, Q), lambda b: (0, b)),
            pl.BlockSpec((2, 2, Q), lambda b: (0, 0, b)),
            pl.BlockSpec((B, C_OCC), lambda b: (0, 0)),
            pl.BlockSpec(memory_space=pltpu.SMEM),
        ],
        out_specs=pl.BlockSpec(memory_space=pltpu.SMEM),
        out_shape=jax.ShapeDtypeStruct((1,), jnp.float32),
        scratch_shapes=[pltpu.SMEM((8,), jnp.float32)],
    )(portion_logits, binary_mask_logits, true_segmap, matched3, inc_t, ie,
      pos_t, chol_t, occupancy_logits, occupancy_target)
    return out[0]
